# Initial kernel scaffold; baseline (speedup 1.0000x reference)
import jax
import jax.numpy as jnp
from jax.experimental import pallas as pl


def kernel(xi, tables):
    raise NotImplementedError("write your pallas kernel here")


# speedup vs baseline: 6.1063x; 6.1063x over previous
import jax
import jax.numpy as jnp
from jax import lax
from jax.experimental import pallas as pl
from jax.experimental.pallas import tpu as pltpu
from jax.experimental.pallas import tpu_sc as plsc

NC, NS, L = 2, 16, 16
NW = NC * NS

B, D = 16384, 128
KQ = 7
RW = B // NW
C = 64
NCHUNK = RW // C
SEG = D // L


def _body(xi_hbm, tab_hbm, out_hbm, idx_v, buf, out_v, sem):
    wid = lax.axis_index("s") * NC + lax.axis_index("c")
    w_base = wid * RW

    pltpu.sync_copy(xi_hbm.at[:, pl.ds(w_base, RW)], idx_v)

    def chunk_body(ci, _):
        copies = [
            pltpu.async_copy(
                tab_hbm.at[k].at[idx_v.at[k, pl.ds(ci * C, C)]],
                buf.at[k],
                sem,
            )
            for k in range(KQ)
        ]
        for c in copies:
            c.wait()

        def sum_body(j, _):
            for s in range(SEG):
                acc = buf[0, j, pl.ds(s * L, L)]
                for k in range(1, KQ):
                    acc = acc + buf[k, j, pl.ds(s * L, L)]
                out_v[j, pl.ds(s * L, L)] = acc
            return 0

        lax.fori_loop(0, C, sum_body, 0)

        pltpu.sync_copy(out_v, out_hbm.at[pl.ds(w_base + ci * C, C)])
        return 0

    lax.fori_loop(0, NCHUNK, chunk_body, 0)


def kernel(xi, tables):
    xi_t = xi.astype(jnp.int32).T

    mesh = plsc.VectorSubcoreMesh(
        core_axis_name="c", subcore_axis_name="s",
        num_cores=NC, num_subcores=NS,
    )
    f = pl.kernel(
        _body,
        out_type=jax.ShapeDtypeStruct((B, D), tables.dtype),
        mesh=mesh,
        scratch_types=[
            pltpu.VMEM((KQ, RW), jnp.int32),
            pltpu.VMEM((KQ, C, D), jnp.float32),
            pltpu.VMEM((C, D), jnp.float32),
            pltpu.SemaphoreType.DMA,
        ],
    )
    return f(xi_t[:KQ], tables)


# speedup vs baseline: 7.5774x; 1.2409x over previous
import jax
import jax.numpy as jnp
from jax import lax
from jax.experimental import pallas as pl
from jax.experimental.pallas import tpu as pltpu
from jax.experimental.pallas import tpu_sc as plsc

NC, NS, L = 2, 16, 16
NW = NC * NS

B, D = 16384, 128
KQ = 7
RW = B // NW


C = 64
NCHUNK = RW // C


def _body(xi_hbm, tab_hbm, out_hbm, idx_v, acc, sem):
    wid = lax.axis_index("s") * NC + lax.axis_index("c")
    w_base = wid * RW

    pltpu.sync_copy(xi_hbm.at[:, pl.ds(w_base, RW)], idx_v)

    first = [
        pltpu.async_copy(
            tab_hbm.at[0].at[idx_v.at[0, pl.ds(ci * C, C)]],
            acc.at[pl.ds(ci * C, C)],
            sem,
        )
        for ci in range(NCHUNK)
    ]
    for c in first:
        c.wait()
    rest = [
        pltpu.async_copy(
            tab_hbm.at[k].at[idx_v.at[k, pl.ds(ci * C, C)]],
            acc.at[pl.ds(ci * C, C)],
            sem,
            add=True,
        )
        for k in range(1, KQ)
        for ci in range(NCHUNK)
    ]
    for c in rest:
        c.wait()

    pltpu.sync_copy(acc, out_hbm.at[pl.ds(w_base, RW)])


def kernel(xi, tables):
    xi_t = xi.astype(jnp.int32).T

    mesh = plsc.VectorSubcoreMesh(
        core_axis_name="c", subcore_axis_name="s",
        num_cores=NC, num_subcores=NS,
    )
    f = pl.kernel(
        _body,
        out_type=jax.ShapeDtypeStruct((B, D), tables.dtype),
        mesh=mesh,
        scratch_types=[
            pltpu.VMEM((KQ, RW), jnp.int32),
            pltpu.VMEM((RW, D), jnp.float32),
            pltpu.SemaphoreType.DMA,
        ],
    )
    return f(xi_t[:KQ], tables)


# speedup vs baseline: 8.5068x; 1.1226x over previous
import jax
import jax.numpy as jnp
from jax import lax
from jax.experimental import pallas as pl
from jax.experimental.pallas import tpu as pltpu
from jax.experimental.pallas import tpu_sc as plsc

NC, NS, L = 2, 16, 16
NW = NC * NS

B, D = 16384, 128
KQ = 7
RW = B // NW
C = 128
NCHUNK = RW // C
IDXW = KQ * RW


def _body(xi_hbm, tab_hbm, out_hbm, idx_v, acc, sem, sem_add):
    wid = lax.axis_index("s") * NC + lax.axis_index("c")
    w_base = wid * RW

    pltpu.sync_copy(xi_hbm.at[pl.ds(wid * IDXW, IDXW)], idx_v)

    first = [
        pltpu.async_copy(
            tab_hbm.at[0].at[idx_v.at[pl.ds(ci * C, C)]],
            acc.at[pl.ds(ci * C, C)],
            sem,
        )
        for ci in range(NCHUNK)
    ]
    rest = []
    for ci in range(NCHUNK):
        first[ci].wait()
        rest += [
            pltpu.async_copy(
                tab_hbm.at[k].at[idx_v.at[pl.ds(k * RW + ci * C, C)]],
                acc.at[pl.ds(ci * C, C)],
                sem_add,
                add=True,
            )
            for k in range(1, KQ)
        ]
    for c in rest:
        c.wait()

    pltpu.sync_copy(acc, out_hbm.at[pl.ds(w_base, RW)])


def kernel(xi, tables):
    xi_t = xi.astype(jnp.int32).T[:KQ]
    xi_w = xi_t.reshape(KQ, NW, RW).transpose(1, 0, 2).reshape(-1)

    mesh = plsc.VectorSubcoreMesh(
        core_axis_name="c", subcore_axis_name="s",
        num_cores=NC, num_subcores=NS,
    )
    f = pl.kernel(
        _body,
        out_type=jax.ShapeDtypeStruct((B, D), tables.dtype),
        mesh=mesh,
        scratch_types=[
            pltpu.VMEM((IDXW,), jnp.int32),
            pltpu.VMEM((RW, D), jnp.float32),
            pltpu.SemaphoreType.DMA,
            pltpu.SemaphoreType.DMA,
        ],
    )
    return f(xi_w, tables)
